# baseline (speedup 1.0000x reference)
import jax
import jax.numpy as jnp
from jax.experimental import pallas as pl
from jax.experimental.pallas import tpu as pltpu

_TOKENS = 16384
_DIM = 2048
_EXPERTS = 64
_BLOCK_T = 2048
_SUB = _BLOCK_T // 2


def _router_body(xa_ref, xb_ref, w_ref, b_ref, out_ref):
    w = w_ref[...]
    bias = b_ref[...].T
    out_ref[:, 0:_SUB] = jax.lax.dot_general(
        w, xa_ref[...],
        dimension_numbers=(((1,), (1,)), ((), ())),
        preferred_element_type=jnp.float32,
    ) + bias
    out_ref[:, _SUB:_BLOCK_T] = jax.lax.dot_general(
        w, xb_ref[...],
        dimension_numbers=(((1,), (1,)), ((), ())),
        preferred_element_type=jnp.float32,
    ) + bias


@jax.jit
def kernel(x, W, b):
    grid = (_TOKENS // _BLOCK_T,)
    out_t = pl.pallas_call(
        _router_body,
        grid=grid,
        in_specs=[
            pl.BlockSpec((_SUB, _DIM), lambda i: (2 * i, 0)),
            pl.BlockSpec((_SUB, _DIM), lambda i: (2 * i + 1, 0)),
            pl.BlockSpec((_EXPERTS, _DIM), lambda i: (0, 0)),
            pl.BlockSpec((1, _EXPERTS), lambda i: (0, 0)),
        ],
        out_specs=pl.BlockSpec((_EXPERTS, _BLOCK_T), lambda i: (0, i)),
        out_shape=jax.ShapeDtypeStruct((_EXPERTS, _TOKENS), jnp.float32),
        compiler_params=pltpu.CompilerParams(
            dimension_semantics=("arbitrary",),
        ),
    )(x, x, W, b.reshape(1, _EXPERTS))
    return out_t.T


# speedup vs baseline: 1.0238x; 1.0238x over previous
import jax
import jax.numpy as jnp
from jax.experimental import pallas as pl
from jax.experimental.pallas import tpu as pltpu

_TOKENS = 16384
_DIM = 2048
_EXPERTS = 64
_BLOCK_T = 1024


def _router_body(x_ref, w_ref, b_ref, out_ref):
    out_ref[...] = jax.lax.dot_general(
        w_ref[...],
        x_ref[...],
        dimension_numbers=(((1,), (1,)), ((), ())),
        preferred_element_type=jnp.float32,
    ) + b_ref[...].T


@jax.jit
def kernel(x, W, b):
    grid = (_TOKENS // _BLOCK_T,)
    out_t = pl.pallas_call(
        _router_body,
        grid=grid,
        in_specs=[
            pl.BlockSpec((_BLOCK_T, _DIM), lambda i: (i, 0)),
            pl.BlockSpec((_EXPERTS, _DIM), lambda i: (0, 0)),
            pl.BlockSpec((1, _EXPERTS), lambda i: (0, 0)),
        ],
        out_specs=pl.BlockSpec((_EXPERTS, _BLOCK_T), lambda i: (0, i)),
        out_shape=jax.ShapeDtypeStruct((_EXPERTS, _TOKENS), jnp.float32),
        compiler_params=pltpu.CompilerParams(
            dimension_semantics=("arbitrary",),
        ),
    )(x, W, b.reshape(1, _EXPERTS))
    return out_t.T
